# baseline (device time: 140102 ns/iter reference)
import jax
import jax.numpy as jnp
from jax import lax
from jax.experimental import pallas as pl
from jax.experimental.pallas import tpu as pltpu

N_DEV = 16
B_PER = 2
SQ = 128
D = 512
H_PER = 8
DH = 64
SCALE = 0.125

N_CW = 15
N_CCW = N_DEV - 1 - N_CW

RING = [0, 1, 2, 3, 7, 6, 5, 9, 10, 11, 15, 14, 13, 12, 8, 4]
POS = [0] * N_DEV
for _p, _m in enumerate(RING):
    POS[_m] = _p


def kernel(x, Wq, Wo, Wk, Wv):
    def body(x_ref, wq_ref, wk_ref, wv_ref, wo_ref, out_ref,
             xg_ref, own_part, rs_send_ref, rs_recv_ref,
             wq16, wk16, wv16, wo16,
             cw_send_sems, cw_recv_sems, ccw_send_sems, ccw_recv_sems,
             rs_send_sems, rs_recv_sems):
        def lookup(table, idx):
            val = jnp.int32(table[0])
            for p in range(1, N_DEV):
                val = jnp.where(idx == p, jnp.int32(table[p]), val)
            return val

        my = lax.axis_index("i")
        pos = lookup(POS, my)

        def mesh_at(ring_pos_offset):
            return lookup(RING, lax.rem(pos + ring_pos_offset + 2 * N_DEV,
                                        N_DEV))

        left = mesh_at(-1)
        right = mesh_at(1)

        barrier = pltpu.get_barrier_semaphore()
        for r in range(1, N_DEV):
            pl.semaphore_signal(barrier, inc=1, device_id=(mesh_at(r),),
                                device_id_type=pl.DeviceIdType.MESH)
        pl.semaphore_wait(barrier, N_DEV - 1)

        def cw_desc(h):
            return pltpu.make_async_remote_copy(
                src_ref=xg_ref.at[h],
                dst_ref=xg_ref.at[h + 1],
                send_sem=cw_send_sems.at[h],
                recv_sem=cw_recv_sems.at[h],
                device_id=(right,),
                device_id_type=pl.DeviceIdType.MESH,
            )

        def ccw_desc(j):
            return pltpu.make_async_remote_copy(
                src_ref=xg_ref.at[(N_DEV - j) % N_DEV],
                dst_ref=xg_ref.at[N_DEV - 1 - j],
                send_sem=ccw_send_sems.at[j],
                recv_sem=ccw_recv_sems.at[j],
                device_id=(left,),
                device_id_type=pl.DeviceIdType.MESH,
            )

        def rs_desc(r):
            return pltpu.make_async_remote_copy(
                src_ref=rs_send_ref.at[r - 1],
                dst_ref=rs_recv_ref.at[r - 1],
                send_sem=rs_send_sems.at[r - 1],
                recv_sem=rs_recv_sems.at[r - 1],
                device_id=(mesh_at(-r),),
                device_id_type=pl.DeviceIdType.MESH,
            )

        cw_descs = [cw_desc(h) for h in range(N_CW)]
        ccw_descs = [ccw_desc(j) for j in range(N_CCW)]
        rs_descs = [rs_desc(r) for r in range(1, N_DEV)]

        def compute_slot(r):
            xb2 = xg_ref[r].reshape(B_PER * SQ, D)
            q = jnp.dot(xb2, wq16[...],
                        preferred_element_type=jnp.float32).astype(jnp.bfloat16)
            k = jnp.dot(xb2, wk16[...],
                        preferred_element_type=jnp.float32).astype(jnp.bfloat16)
            v = jnp.dot(xb2, wv16[...],
                        preferred_element_type=jnp.float32).astype(jnp.bfloat16)
            o_rows = []
            for b in range(B_PER):
                rsl = slice(b * SQ, (b + 1) * SQ)
                o_cols = []
                for hh in range(H_PER):
                    csl = slice(hh * DH, (hh + 1) * DH)
                    qh, kh, vh = q[rsl, csl], k[rsl, csl], v[rsl, csl]
                    s = lax.dot_general(
                        qh, kh, (((1,), (1,)), ((), ())),
                        preferred_element_type=jnp.float32) * SCALE
                    m = jnp.max(s, axis=1, keepdims=True)
                    e = jnp.exp(s - m)
                    pmat = (e / jnp.sum(e, axis=1, keepdims=True)).astype(jnp.bfloat16)
                    o_cols.append(jnp.dot(pmat, vh,
                                          preferred_element_type=jnp.float32))
                o_rows.append(jnp.concatenate(o_cols, axis=1))
            attn = jnp.concatenate(o_rows, axis=0).astype(jnp.bfloat16)
            part = jnp.dot(attn, wo16[...],
                           preferred_element_type=jnp.float32
                           ).reshape(B_PER, SQ, D)
            if r == 0:
                own_part[...] = part
            else:
                rs_send_ref[r - 1] = part.astype(jnp.bfloat16)
                rs_descs[r - 1].start()

        wq16[...] = wq_ref[...].astype(jnp.bfloat16)
        wk16[...] = wk_ref[...].astype(jnp.bfloat16)
        wv16[...] = wv_ref[...].astype(jnp.bfloat16)
        wo16[...] = wo_ref[...].astype(jnp.bfloat16)
        xg_ref[0] = x_ref[...].astype(jnp.bfloat16)

        cw_descs[0].start()
        if N_CCW:
            ccw_descs[0].start()
        compute_slot(0)
        for k_step in range(1, N_CW + 1):
            cw_descs[k_step - 1].wait_recv()
            if k_step < N_CW:
                cw_descs[k_step].start()
            compute_slot(k_step)
            if k_step <= N_CCW:
                ccw_descs[k_step - 1].wait_recv()
                if k_step < N_CCW:
                    ccw_descs[k_step].start()
                compute_slot(N_DEV - k_step)

        acc = own_part[...]
        for j in range(N_DEV - 1):
            rs_descs[j].wait_recv()
            acc = acc + rs_recv_ref[j].astype(jnp.float32)
        out_ref[...] = acc

        for d in cw_descs + ccw_descs + rs_descs:
            d.wait_send()

    return pl.pallas_call(
        body,
        out_shape=jax.ShapeDtypeStruct((B_PER, SQ, D), jnp.float32),
        in_specs=[pl.BlockSpec(memory_space=pltpu.VMEM)] * 5,
        out_specs=pl.BlockSpec(memory_space=pltpu.VMEM),
        scratch_shapes=[
            pltpu.VMEM((N_DEV, B_PER, SQ, D), jnp.bfloat16),
            pltpu.VMEM((B_PER, SQ, D), jnp.float32),
            pltpu.VMEM((N_DEV - 1, B_PER, SQ, D), jnp.bfloat16),
            pltpu.VMEM((N_DEV - 1, B_PER, SQ, D), jnp.bfloat16),
            pltpu.VMEM((D, D), jnp.bfloat16),
            pltpu.VMEM((D, D), jnp.bfloat16),
            pltpu.VMEM((D, D), jnp.bfloat16),
            pltpu.VMEM((D, D), jnp.bfloat16),
            pltpu.SemaphoreType.DMA((N_CW,)),
            pltpu.SemaphoreType.DMA((N_CW,)),
            pltpu.SemaphoreType.DMA((max(N_CCW, 1),)),
            pltpu.SemaphoreType.DMA((max(N_CCW, 1),)),
            pltpu.SemaphoreType.DMA((N_DEV - 1,)),
            pltpu.SemaphoreType.DMA((N_DEV - 1,)),
        ],
        compiler_params=pltpu.CompilerParams(collective_id=0),
    )(x, Wq, Wk, Wv, Wo)


# device time: 120055 ns/iter; 1.1670x vs baseline; 1.1670x over previous
import jax
import jax.numpy as jnp
from jax import lax
from jax.experimental import pallas as pl
from jax.experimental.pallas import tpu as pltpu

N_DEV = 16
B_PER = 2
SQ = 128
D = 512
H_PER = 8
DH = 64
SCALE = 0.125

N_CW = 8
N_CCW = 7

RING = [0, 1, 2, 3, 7, 6, 5, 9, 10, 11, 15, 14, 13, 12, 8, 4]
POS = [0] * N_DEV
for _p, _m in enumerate(RING):
    POS[_m] = _p

F32 = jnp.float32
BF16 = jnp.bfloat16


def kernel(x, Wq, Wo, Wk, Wv):
    def body(x_ref, wq_ref, wk_ref, wv_ref, wo_ref, out_ref,
             xg_ref, part_ref,
             asend_ref, arecv_ref, bsend_ref, brecv_ref,
             afin_ref, bfin_ref, adr_ref, bdr_ref,
             wq16, wk16, wv16, wo16,
             cw_s, cw_r, ccw_s, ccw_r,
             a_s, a_r, b_s, b_r,
             afin_s, afin_r, bfin_s, bfin_r):
        def lookup(table, idx):
            val = jnp.int32(table[0])
            for p in range(1, N_DEV):
                val = jnp.where(idx == p, jnp.int32(table[p]), val)
            return val

        my = lax.axis_index("i")
        pos = lookup(POS, my)

        def mesh_at(off):
            return lookup(RING, lax.rem(pos + off + 2 * N_DEV, N_DEV))

        left = mesh_at(-1)
        right = mesh_at(1)

        wq16[...] = wq_ref[...].astype(BF16)
        wk16[...] = wk_ref[...].astype(BF16)
        wv16[...] = wv_ref[...].astype(BF16)
        wo16[...] = wo_ref[...].astype(BF16)

        barrier = pltpu.get_barrier_semaphore()
        for r in range(1, N_DEV):
            pl.semaphore_signal(barrier, inc=1, device_id=(mesh_at(r),),
                                device_id_type=pl.DeviceIdType.MESH)
        pl.semaphore_wait(barrier, N_DEV - 1)

        def cw_desc(h):
            return pltpu.make_async_remote_copy(
                src_ref=xg_ref.at[h], dst_ref=xg_ref.at[h + 1],
                send_sem=cw_s.at[h], recv_sem=cw_r.at[h],
                device_id=(right,), device_id_type=pl.DeviceIdType.MESH)

        def ccw_desc(j):
            return pltpu.make_async_remote_copy(
                src_ref=xg_ref.at[(N_DEV - j) % N_DEV],
                dst_ref=xg_ref.at[N_DEV - 1 - j],
                send_sem=ccw_s.at[j], recv_sem=ccw_r.at[j],
                device_id=(left,), device_id_type=pl.DeviceIdType.MESH)

        def a_desc(i):
            return pltpu.make_async_remote_copy(
                src_ref=asend_ref.at[i], dst_ref=arecv_ref.at[i],
                send_sem=a_s.at[i], recv_sem=a_r.at[i],
                device_id=(right,), device_id_type=pl.DeviceIdType.MESH)

        def b_desc(i):
            return pltpu.make_async_remote_copy(
                src_ref=bsend_ref.at[i], dst_ref=brecv_ref.at[i],
                send_sem=b_s.at[i], recv_sem=b_r.at[i],
                device_id=(left,), device_id_type=pl.DeviceIdType.MESH)

        cw_descs = [cw_desc(h) for h in range(N_CW)]
        ccw_descs = [ccw_desc(j) for j in range(N_CCW)]
        a_descs = [a_desc(i) for i in range(6)]
        b_descs = [b_desc(i) for i in range(7)]
        afin_desc = pltpu.make_async_remote_copy(
            src_ref=afin_ref, dst_ref=adr_ref,
            send_sem=afin_s, recv_sem=afin_r,
            device_id=(mesh_at(-7),), device_id_type=pl.DeviceIdType.MESH)
        bfin_desc = pltpu.make_async_remote_copy(
            src_ref=bfin_ref, dst_ref=bdr_ref,
            send_sem=bfin_s, recv_sem=bfin_r,
            device_id=(mesh_at(8),), device_id_type=pl.DeviceIdType.MESH)

        def compute_slot(r):
            xb2 = xg_ref[r].reshape(B_PER * SQ, D)
            q = jnp.dot(xb2, wq16[...],
                        preferred_element_type=F32).astype(BF16)
            k = jnp.dot(xb2, wk16[...],
                        preferred_element_type=F32).astype(BF16)
            v = jnp.dot(xb2, wv16[...],
                        preferred_element_type=F32).astype(BF16)
            o_rows = []
            for b in range(B_PER):
                rsl = slice(b * SQ, (b + 1) * SQ)
                o_cols = []
                for hh in range(H_PER):
                    csl = slice(hh * DH, (hh + 1) * DH)
                    qh, kh, vh = q[rsl, csl], k[rsl, csl], v[rsl, csl]
                    s = lax.dot_general(
                        qh, kh, (((1,), (1,)), ((), ())),
                        preferred_element_type=F32) * SCALE
                    m = jnp.max(s, axis=1, keepdims=True)
                    e = jnp.exp(s - m)
                    pmat = (e / jnp.sum(e, axis=1, keepdims=True)).astype(BF16)
                    o_cols.append(jnp.dot(pmat, vh, preferred_element_type=F32))
                o_rows.append(jnp.concatenate(o_cols, axis=1))
            attn = jnp.concatenate(o_rows, axis=0).astype(BF16)
            part_ref[r] = jnp.dot(attn, wo16[...],
                                  preferred_element_type=F32
                                  ).reshape(B_PER, SQ, D)

        xg_ref[0] = x_ref[...].astype(BF16)
        cw_descs[0].start()
        ccw_descs[0].start()
        compute_slot(0)

        for d in range(1, N_CW + 1):
            cw_descs[d - 1].wait_recv()
            if d < N_CW:
                cw_descs[d].start()
            compute_slot(d)
            if d <= 6:
                if d >= 2:
                    a_descs[d - 2].wait_recv()
                    acc = part_ref[d] + arecv_ref[d - 2].astype(F32)
                else:
                    acc = part_ref[1]
                asend_ref[d - 1] = acc.astype(BF16)
                a_descs[d - 1].start()
            elif d == 7:
                a_descs[5].wait_recv()
                afin_ref[...] = (part_ref[7]
                                 + arecv_ref[5].astype(F32)).astype(BF16)
                afin_desc.start()

            if d <= N_CCW:
                ccw_descs[d - 1].wait_recv()
                if d < N_CCW:
                    ccw_descs[d].start()
                compute_slot(N_DEV - d)
                if d >= 2:
                    b_descs[d - 2].wait_recv()
                    accb = part_ref[N_DEV - d] + brecv_ref[d - 2].astype(F32)
                else:
                    accb = part_ref[N_DEV - 1]
                bsend_ref[d - 1] = accb.astype(BF16)
                b_descs[d - 1].start()
            else:
                b_descs[6].wait_recv()
                bfin_ref[...] = (part_ref[8]
                                 + brecv_ref[6].astype(F32)).astype(BF16)
                bfin_desc.start()

        afin_desc.wait_recv()
        bfin_desc.wait_recv()
        out_ref[...] = (part_ref[0] + adr_ref[...].astype(F32)
                        + bdr_ref[...].astype(F32))

        for dsc in cw_descs + ccw_descs + a_descs + b_descs:
            dsc.wait_send()
        afin_desc.wait_send()
        bfin_desc.wait_send()

    chunk = (B_PER, SQ, D)
    return pl.pallas_call(
        body,
        out_shape=jax.ShapeDtypeStruct(chunk, F32),
        in_specs=[pl.BlockSpec(memory_space=pltpu.VMEM)] * 5,
        out_specs=pl.BlockSpec(memory_space=pltpu.VMEM),
        scratch_shapes=[
            pltpu.VMEM((N_DEV,) + chunk, BF16),
            pltpu.VMEM((N_DEV,) + chunk, F32),
            pltpu.VMEM((6,) + chunk, BF16),
            pltpu.VMEM((6,) + chunk, BF16),
            pltpu.VMEM((7,) + chunk, BF16),
            pltpu.VMEM((7,) + chunk, BF16),
            pltpu.VMEM(chunk, BF16),
            pltpu.VMEM(chunk, BF16),
            pltpu.VMEM(chunk, BF16),
            pltpu.VMEM(chunk, BF16),
            pltpu.VMEM((D, D), BF16),
            pltpu.VMEM((D, D), BF16),
            pltpu.VMEM((D, D), BF16),
            pltpu.VMEM((D, D), BF16),
            pltpu.SemaphoreType.DMA((N_CW,)),
            pltpu.SemaphoreType.DMA((N_CW,)),
            pltpu.SemaphoreType.DMA((N_CCW,)),
            pltpu.SemaphoreType.DMA((N_CCW,)),
            pltpu.SemaphoreType.DMA((6,)),
            pltpu.SemaphoreType.DMA((6,)),
            pltpu.SemaphoreType.DMA((7,)),
            pltpu.SemaphoreType.DMA((7,)),
            pltpu.SemaphoreType.DMA,
            pltpu.SemaphoreType.DMA,
            pltpu.SemaphoreType.DMA,
            pltpu.SemaphoreType.DMA,
        ],
        compiler_params=pltpu.CompilerParams(collective_id=0),
    )(x, Wq, Wk, Wv, Wo)


# device time: 72695 ns/iter; 1.9273x vs baseline; 1.6515x over previous
import jax
import jax.numpy as jnp
from jax import lax
from jax.experimental import pallas as pl
from jax.experimental.pallas import tpu as pltpu

N_DEV = 16
B_PER = 2
SQ = 128
D = 512
H_PER = 8
DH = 64
SCALE = 0.125

N_CW = 8
N_CCW = 7

RING = [0, 1, 2, 3, 7, 6, 5, 9, 10, 11, 15, 14, 13, 12, 8, 4]
POS = [0] * N_DEV
for _p, _m in enumerate(RING):
    POS[_m] = _p

F32 = jnp.float32
BF16 = jnp.bfloat16


def kernel(x, Wq, Wo, Wk, Wv):
    def body(x_ref, wq_ref, wk_ref, wv_ref, wo_ref, out_ref,
             xg_ref, part_ref,
             asend_ref, arecv_ref, bsend_ref, brecv_ref,
             afin_ref, bfin_ref, adr_ref, bdr_ref,
             wq16, wk16, wv16, wo16,
             cw_s, cw_r, ccw_s, ccw_r,
             a_s, a_r, b_s, b_r,
             afin_s, afin_r, bfin_s, bfin_r):
        def lookup(table, idx):
            val = jnp.int32(table[0])
            for p in range(1, N_DEV):
                val = jnp.where(idx == p, jnp.int32(table[p]), val)
            return val

        my = lax.axis_index("i")
        pos = lookup(POS, my)

        def mesh_at(off):
            return lookup(RING, lax.rem(pos + off + 2 * N_DEV, N_DEV))

        left = mesh_at(-1)
        right = mesh_at(1)

        wq16[...] = wq_ref[...].astype(BF16)
        wk16[...] = wk_ref[...].astype(BF16)
        wv16[...] = wv_ref[...].astype(BF16)
        wo16[...] = wo_ref[...].astype(BF16)

        barrier = pltpu.get_barrier_semaphore()
        for r in range(1, N_DEV):
            pl.semaphore_signal(barrier, inc=1, device_id=(mesh_at(r),),
                                device_id_type=pl.DeviceIdType.MESH)
        pl.semaphore_wait(barrier, N_DEV - 1)

        def cw_desc(h):
            return pltpu.make_async_remote_copy(
                src_ref=xg_ref.at[h], dst_ref=xg_ref.at[h + 1],
                send_sem=cw_s.at[h], recv_sem=cw_r.at[h],
                device_id=(right,), device_id_type=pl.DeviceIdType.MESH)

        def ccw_desc(j):
            return pltpu.make_async_remote_copy(
                src_ref=xg_ref.at[(N_DEV - j) % N_DEV],
                dst_ref=xg_ref.at[N_DEV - 1 - j],
                send_sem=ccw_s.at[j], recv_sem=ccw_r.at[j],
                device_id=(left,), device_id_type=pl.DeviceIdType.MESH)

        def a_desc(i):
            return pltpu.make_async_remote_copy(
                src_ref=asend_ref.at[i], dst_ref=arecv_ref.at[i],
                send_sem=a_s.at[i], recv_sem=a_r.at[i],
                device_id=(right,), device_id_type=pl.DeviceIdType.MESH)

        def b_desc(i):
            return pltpu.make_async_remote_copy(
                src_ref=bsend_ref.at[i], dst_ref=brecv_ref.at[i],
                send_sem=b_s.at[i], recv_sem=b_r.at[i],
                device_id=(left,), device_id_type=pl.DeviceIdType.MESH)

        cw_descs = [cw_desc(h) for h in range(N_CW)]
        ccw_descs = [ccw_desc(j) for j in range(N_CCW)]
        a_descs = [a_desc(i) for i in range(6)]
        b_descs = [b_desc(i) for i in range(7)]
        afin_desc = pltpu.make_async_remote_copy(
            src_ref=afin_ref, dst_ref=adr_ref,
            send_sem=afin_s, recv_sem=afin_r,
            device_id=(mesh_at(-7),), device_id_type=pl.DeviceIdType.MESH)
        bfin_desc = pltpu.make_async_remote_copy(
            src_ref=bfin_ref, dst_ref=bdr_ref,
            send_sem=bfin_s, recv_sem=bfin_r,
            device_id=(mesh_at(8),), device_id_type=pl.DeviceIdType.MESH)

        def compute_slot(r):
            xb2 = xg_ref[r].reshape(B_PER * SQ, D)
            q = jnp.dot(xb2, wq16[...],
                        preferred_element_type=F32).astype(BF16)
            k = jnp.dot(xb2, wk16[...],
                        preferred_element_type=F32).astype(BF16)
            v = jnp.dot(xb2, wv16[...],
                        preferred_element_type=F32).astype(BF16)
            o_rows = []
            for b in range(B_PER):
                rsl = slice(b * SQ, (b + 1) * SQ)
                q3 = q[rsl].reshape(SQ, H_PER, DH).transpose(1, 0, 2)
                k3 = k[rsl].reshape(SQ, H_PER, DH).transpose(1, 0, 2)
                v3 = v[rsl].reshape(SQ, H_PER, DH).transpose(1, 0, 2)
                s3 = lax.dot_general(
                    q3, k3, (((2,), (2,)), ((0,), (0,))),
                    preferred_element_type=F32) * SCALE
                m = jnp.max(s3, axis=-1, keepdims=True)
                e = jnp.exp(s3 - m)
                p3 = (e / jnp.sum(e, axis=-1, keepdims=True)).astype(BF16)
                o3 = lax.dot_general(
                    p3, v3, (((2,), (1,)), ((0,), (0,))),
                    preferred_element_type=F32)
                o_rows.append(o3.transpose(1, 0, 2).reshape(SQ, H_PER * DH))
            attn = jnp.concatenate(o_rows, axis=0).astype(BF16)
            part_ref[r] = jnp.dot(attn, wo16[...],
                                  preferred_element_type=F32
                                  ).reshape(B_PER, SQ, D)

        xg_ref[0] = x_ref[...].astype(BF16)
        cw_descs[0].start()
        ccw_descs[0].start()
        compute_slot(0)

        for d in range(1, N_CW + 1):
            cw_descs[d - 1].wait_recv()
            if d < N_CW:
                cw_descs[d].start()
            compute_slot(d)
            if d <= 6:
                if d >= 2:
                    a_descs[d - 2].wait_recv()
                    acc = part_ref[d] + arecv_ref[d - 2].astype(F32)
                else:
                    acc = part_ref[1]
                asend_ref[d - 1] = acc.astype(BF16)
                a_descs[d - 1].start()
            elif d == 7:
                a_descs[5].wait_recv()
                afin_ref[...] = (part_ref[7]
                                 + arecv_ref[5].astype(F32)).astype(BF16)
                afin_desc.start()

            if d <= N_CCW:
                ccw_descs[d - 1].wait_recv()
                if d < N_CCW:
                    ccw_descs[d].start()
                compute_slot(N_DEV - d)
                if d >= 2:
                    b_descs[d - 2].wait_recv()
                    accb = part_ref[N_DEV - d] + brecv_ref[d - 2].astype(F32)
                else:
                    accb = part_ref[N_DEV - 1]
                bsend_ref[d - 1] = accb.astype(BF16)
                b_descs[d - 1].start()
            else:
                b_descs[6].wait_recv()
                bfin_ref[...] = (part_ref[8]
                                 + brecv_ref[6].astype(F32)).astype(BF16)
                bfin_desc.start()

        afin_desc.wait_recv()
        bfin_desc.wait_recv()
        out_ref[...] = (part_ref[0] + adr_ref[...].astype(F32)
                        + bdr_ref[...].astype(F32))

        for dsc in cw_descs + ccw_descs + a_descs + b_descs:
            dsc.wait_send()
        afin_desc.wait_send()
        bfin_desc.wait_send()

    chunk = (B_PER, SQ, D)
    return pl.pallas_call(
        body,
        out_shape=jax.ShapeDtypeStruct(chunk, F32),
        in_specs=[pl.BlockSpec(memory_space=pltpu.VMEM)] * 5,
        out_specs=pl.BlockSpec(memory_space=pltpu.VMEM),
        scratch_shapes=[
            pltpu.VMEM((N_DEV,) + chunk, BF16),
            pltpu.VMEM((N_DEV,) + chunk, F32),
            pltpu.VMEM((6,) + chunk, BF16),
            pltpu.VMEM((6,) + chunk, BF16),
            pltpu.VMEM((7,) + chunk, BF16),
            pltpu.VMEM((7,) + chunk, BF16),
            pltpu.VMEM(chunk, BF16),
            pltpu.VMEM(chunk, BF16),
            pltpu.VMEM(chunk, BF16),
            pltpu.VMEM(chunk, BF16),
            pltpu.VMEM((D, D), BF16),
            pltpu.VMEM((D, D), BF16),
            pltpu.VMEM((D, D), BF16),
            pltpu.VMEM((D, D), BF16),
            pltpu.SemaphoreType.DMA((N_CW,)),
            pltpu.SemaphoreType.DMA((N_CW,)),
            pltpu.SemaphoreType.DMA((N_CCW,)),
            pltpu.SemaphoreType.DMA((N_CCW,)),
            pltpu.SemaphoreType.DMA((6,)),
            pltpu.SemaphoreType.DMA((6,)),
            pltpu.SemaphoreType.DMA((7,)),
            pltpu.SemaphoreType.DMA((7,)),
            pltpu.SemaphoreType.DMA,
            pltpu.SemaphoreType.DMA,
            pltpu.SemaphoreType.DMA,
            pltpu.SemaphoreType.DMA,
        ],
        compiler_params=pltpu.CompilerParams(collective_id=0),
    )(x, Wq, Wk, Wv, Wo)


# device time: 72632 ns/iter; 1.9289x vs baseline; 1.0009x over previous
import jax
import jax.numpy as jnp
from jax import lax
from jax.experimental import pallas as pl
from jax.experimental.pallas import tpu as pltpu

N_DEV = 16
B_PER = 2
SQ = 128
D = 512
H_PER = 8
DH = 64
SCALE = 0.125

N_CW = 8
N_CCW = 7

RING = [0, 1, 2, 3, 7, 6, 5, 9, 10, 11, 15, 14, 13, 12, 8, 4]
POS = [0] * N_DEV
for _p, _m in enumerate(RING):
    POS[_m] = _p

F32 = jnp.float32
BF16 = jnp.bfloat16


def kernel(x, Wq, Wo, Wk, Wv):
    def body(x_ref, wq_ref, wk_ref, wv_ref, wo_ref, out_ref,
             xg_ref, part_ref,
             asend_ref, arecv_ref, bsend_ref, brecv_ref,
             afin_ref, bfin_ref, adr_ref, bdr_ref,
             wq16, wk16, wv16, wo16,
             cw_s, cw_r, ccw_s, ccw_r,
             a_s, a_r, b_s, b_r,
             afin_s, afin_r, bfin_s, bfin_r):
        def lookup(table, idx):
            val = jnp.int32(table[0])
            for p in range(1, N_DEV):
                val = jnp.where(idx == p, jnp.int32(table[p]), val)
            return val

        my = lax.axis_index("i")
        pos = lookup(POS, my)

        def mesh_at(off):
            return lookup(RING, lax.rem(pos + off + 2 * N_DEV, N_DEV))

        left = mesh_at(-1)
        right = mesh_at(1)

        wq16[...] = wq_ref[...].astype(BF16)
        wk16[...] = wk_ref[...].astype(BF16)
        wv16[...] = wv_ref[...].astype(BF16)
        wo16[...] = wo_ref[...].astype(BF16)

        barrier = pltpu.get_barrier_semaphore()
        for r in range(1, N_DEV):
            pl.semaphore_signal(barrier, inc=1, device_id=(mesh_at(r),),
                                device_id_type=pl.DeviceIdType.MESH)
        pl.semaphore_wait(barrier, N_DEV - 1)

        def cw_desc(h):
            return pltpu.make_async_remote_copy(
                src_ref=xg_ref.at[h], dst_ref=xg_ref.at[h + 1],
                send_sem=cw_s.at[h], recv_sem=cw_r.at[h],
                device_id=(right,), device_id_type=pl.DeviceIdType.MESH)

        def ccw_desc(j):
            return pltpu.make_async_remote_copy(
                src_ref=xg_ref.at[(N_DEV - j) % N_DEV],
                dst_ref=xg_ref.at[N_DEV - 1 - j],
                send_sem=ccw_s.at[j], recv_sem=ccw_r.at[j],
                device_id=(left,), device_id_type=pl.DeviceIdType.MESH)

        def a_desc(i):
            return pltpu.make_async_remote_copy(
                src_ref=asend_ref.at[i], dst_ref=arecv_ref.at[i],
                send_sem=a_s.at[i], recv_sem=a_r.at[i],
                device_id=(right,), device_id_type=pl.DeviceIdType.MESH)

        def b_desc(i):
            return pltpu.make_async_remote_copy(
                src_ref=bsend_ref.at[i], dst_ref=brecv_ref.at[i],
                send_sem=b_s.at[i], recv_sem=b_r.at[i],
                device_id=(left,), device_id_type=pl.DeviceIdType.MESH)

        cw_descs = [cw_desc(h) for h in range(N_CW)]
        ccw_descs = [ccw_desc(j) for j in range(N_CCW)]
        a_descs = [a_desc(i) for i in range(6)]
        b_descs = [b_desc(i) for i in range(7)]
        afin_desc = pltpu.make_async_remote_copy(
            src_ref=afin_ref, dst_ref=adr_ref,
            send_sem=afin_s, recv_sem=afin_r,
            device_id=(mesh_at(-7),), device_id_type=pl.DeviceIdType.MESH)
        bfin_desc = pltpu.make_async_remote_copy(
            src_ref=bfin_ref, dst_ref=bdr_ref,
            send_sem=bfin_s, recv_sem=bfin_r,
            device_id=(mesh_at(8),), device_id_type=pl.DeviceIdType.MESH)

        def compute_slot(r):
            xb2 = xg_ref[r].reshape(B_PER * SQ, D)
            q = jnp.dot(xb2, wq16[...],
                        preferred_element_type=F32).astype(BF16)
            k = jnp.dot(xb2, wk16[...],
                        preferred_element_type=F32).astype(BF16)
            v = jnp.dot(xb2, wv16[...],
                        preferred_element_type=F32).astype(BF16)
            def to_hsd(t):
                blocks = [t[b * SQ:(b + 1) * SQ].reshape(SQ, H_PER, DH)
                          .transpose(1, 0, 2) for b in range(B_PER)]
                return jnp.concatenate(blocks, axis=0)

            q3, k3, v3 = to_hsd(q), to_hsd(k), to_hsd(v)
            s3 = lax.dot_general(
                q3, k3, (((2,), (2,)), ((0,), (0,))),
                preferred_element_type=F32) * SCALE
            m = jnp.max(s3, axis=-1, keepdims=True)
            e = jnp.exp(s3 - m)
            p3 = (e / jnp.sum(e, axis=-1, keepdims=True)).astype(BF16)
            o3 = lax.dot_general(
                p3, v3, (((2,), (1,)), ((0,), (0,))),
                preferred_element_type=F32)
            o_rows = [o3[b * H_PER:(b + 1) * H_PER].transpose(1, 0, 2)
                      .reshape(SQ, H_PER * DH) for b in range(B_PER)]
            attn = jnp.concatenate(o_rows, axis=0).astype(BF16)
            part_ref[r] = jnp.dot(attn, wo16[...],
                                  preferred_element_type=F32
                                  ).reshape(B_PER, SQ, D)

        xg_ref[0] = x_ref[...].astype(BF16)
        cw_descs[0].start()
        ccw_descs[0].start()
        compute_slot(0)

        for d in range(1, N_CW + 1):
            cw_descs[d - 1].wait_recv()
            if d < N_CW:
                cw_descs[d].start()
            compute_slot(d)
            if d <= 6:
                if d >= 2:
                    a_descs[d - 2].wait_recv()
                    acc = part_ref[d] + arecv_ref[d - 2].astype(F32)
                else:
                    acc = part_ref[1]
                asend_ref[d - 1] = acc.astype(BF16)
                a_descs[d - 1].start()
            elif d == 7:
                a_descs[5].wait_recv()
                afin_ref[...] = (part_ref[7]
                                 + arecv_ref[5].astype(F32)).astype(BF16)
                afin_desc.start()

            if d <= N_CCW:
                ccw_descs[d - 1].wait_recv()
                if d < N_CCW:
                    ccw_descs[d].start()
                compute_slot(N_DEV - d)
                if d >= 2:
                    b_descs[d - 2].wait_recv()
                    accb = part_ref[N_DEV - d] + brecv_ref[d - 2].astype(F32)
                else:
                    accb = part_ref[N_DEV - 1]
                bsend_ref[d - 1] = accb.astype(BF16)
                b_descs[d - 1].start()
            else:
                b_descs[6].wait_recv()
                bfin_ref[...] = (part_ref[8]
                                 + brecv_ref[6].astype(F32)).astype(BF16)
                bfin_desc.start()

        afin_desc.wait_recv()
        bfin_desc.wait_recv()
        out_ref[...] = (part_ref[0] + adr_ref[...].astype(F32)
                        + bdr_ref[...].astype(F32))

        for dsc in cw_descs + ccw_descs + a_descs + b_descs:
            dsc.wait_send()
        afin_desc.wait_send()
        bfin_desc.wait_send()

    chunk = (B_PER, SQ, D)
    return pl.pallas_call(
        body,
        out_shape=jax.ShapeDtypeStruct(chunk, F32),
        in_specs=[pl.BlockSpec(memory_space=pltpu.VMEM)] * 5,
        out_specs=pl.BlockSpec(memory_space=pltpu.VMEM),
        scratch_shapes=[
            pltpu.VMEM((N_DEV,) + chunk, BF16),
            pltpu.VMEM((N_DEV,) + chunk, F32),
            pltpu.VMEM((6,) + chunk, BF16),
            pltpu.VMEM((6,) + chunk, BF16),
            pltpu.VMEM((7,) + chunk, BF16),
            pltpu.VMEM((7,) + chunk, BF16),
            pltpu.VMEM(chunk, BF16),
            pltpu.VMEM(chunk, BF16),
            pltpu.VMEM(chunk, BF16),
            pltpu.VMEM(chunk, BF16),
            pltpu.VMEM((D, D), BF16),
            pltpu.VMEM((D, D), BF16),
            pltpu.VMEM((D, D), BF16),
            pltpu.VMEM((D, D), BF16),
            pltpu.SemaphoreType.DMA((N_CW,)),
            pltpu.SemaphoreType.DMA((N_CW,)),
            pltpu.SemaphoreType.DMA((N_CCW,)),
            pltpu.SemaphoreType.DMA((N_CCW,)),
            pltpu.SemaphoreType.DMA((6,)),
            pltpu.SemaphoreType.DMA((6,)),
            pltpu.SemaphoreType.DMA((7,)),
            pltpu.SemaphoreType.DMA((7,)),
            pltpu.SemaphoreType.DMA,
            pltpu.SemaphoreType.DMA,
            pltpu.SemaphoreType.DMA,
            pltpu.SemaphoreType.DMA,
        ],
        compiler_params=pltpu.CompilerParams(collective_id=0),
    )(x, Wq, Wk, Wv, Wo)
